# Initial kernel scaffold; baseline (speedup 1.0000x reference)
import jax
import jax.numpy as jnp
from jax.experimental import pallas as pl


def kernel(boxes, scores):
    raise NotImplementedError("write your pallas kernel here")


# speedup vs baseline: 286.3624x; 286.3624x over previous
import jax
import jax.numpy as jnp
from jax.experimental import pallas as pl
from jax.experimental.pallas import tpu as pltpu

_N = 5000
_B = 256
_NP = 5120
_NB = _NP // _B
_THR = 0.65
_STRIDE = 32.0


def _decode_rows(raw):
    cx = raw[0:1, :] * 64.0 + 512.0
    cy = raw[1:2, :] * 64.0 + 512.0
    w = jnp.exp(jnp.clip(raw[2:3, :], -4.0, 4.0)) * _STRIDE
    h = jnp.exp(jnp.clip(raw[3:4, :], -4.0, 4.0)) * _STRIDE
    x1 = cx - w * 0.5
    y1 = cy - h * 0.5
    x2 = cx + w * 0.5
    y2 = cy + h * 0.5
    return x1, y1, x2, y2, (x2 - x1) * (y2 - y1)


def _decode_cols(raw):
    cx = raw[:, 0:1] * 64.0 + 512.0
    cy = raw[:, 1:2] * 64.0 + 512.0
    w = jnp.exp(jnp.clip(raw[:, 2:3], -4.0, 4.0)) * _STRIDE
    h = jnp.exp(jnp.clip(raw[:, 3:4], -4.0, 4.0)) * _STRIDE
    x1 = cx - w * 0.5
    y1 = cy - h * 0.5
    x2 = cx + w * 0.5
    y2 = cy + h * 0.5
    return x1, y1, x2, y2, (x2 - x1) * (y2 - y1)


def _iou(bx1, by1, bx2, by2, ba, x1, y1, x2, y2, area):
    ix1 = jnp.maximum(bx1, x1)
    iy1 = jnp.maximum(by1, y1)
    ix2 = jnp.minimum(bx2, x2)
    iy2 = jnp.minimum(by2, y2)
    inter = jnp.maximum(ix2 - ix1, 0.0) * jnp.maximum(iy2 - iy1, 0.0)
    return inter / (ba + area - inter + 1e-7)


def _nms_kernel(raw_t_ref, raw_ref, p_ref, out_ref, alive_ref):
    raw_t = raw_t_ref[...]
    p = p_ref[...]

    x1, y1, x2, y2, area = _decode_rows(raw_t)
    colg = jax.lax.broadcasted_iota(jnp.int32, (1, _NP), 1)
    rowi = jax.lax.broadcasted_iota(jnp.int32, (_B, _B), 0)
    coli = jax.lax.broadcasted_iota(jnp.int32, (_B, _B), 1)
    tri = (rowi < coli).astype(jnp.float32)

    alive_ref[...] = jnp.ones((1, _NP), jnp.float32)

    def block_body(i, carry):
        off = i * _B
        blk = raw_ref[pl.ds(off, _B), :]
        bx1, by1, bx2, by2, ba = _decode_cols(blk)
        blk_t = raw_t_ref[:, pl.ds(off, _B)]
        rx1, ry1, rx2, ry2, rarea = _decode_rows(blk_t)

        supf = (_iou(bx1, by1, bx2, by2, ba,
                     x1, y1, x2, y2, area) > _THR).astype(jnp.float32)
        mf = (_iou(bx1, by1, bx2, by2, ba,
                   rx1, ry1, rx2, ry2, rarea) > _THR).astype(jnp.float32)
        mf = mf * tri

        blk_alive = alive_ref[:, pl.ds(off, _B)]

        def fix_cond(c):
            return c[1]

        def fix_body(c):
            k, _ = c
            hit = jnp.dot(k, mf, preferred_element_type=jnp.float32)
            k_new = blk_alive * (hit <= 0.0).astype(jnp.float32)
            return k_new, jnp.any(k_new != k)

        k, _ = jax.lax.while_loop(fix_cond, fix_body,
                                  (blk_alive, jnp.bool_(True)))

        cross = jnp.dot(k, supf, preferred_element_type=jnp.float32)
        later = (colg >= off + _B).astype(jnp.float32)
        alive_ref[...] = alive_ref[...] * (
            1.0 - later * (cross > 0.0).astype(jnp.float32))
        alive_ref[:, pl.ds(off, _B)] = k
        return carry

    jax.lax.fori_loop(0, _NB, block_body, 0)

    keep = alive_ref[...]
    zeros = jnp.zeros((3, _NP), jnp.float32)
    out_ref[...] = jnp.concatenate(
        [x1 * keep, y1 * keep, x2 * keep, y2 * keep, p * keep, zeros], axis=0)


def kernel(boxes, scores):
    n = boxes.shape[0]
    probs = jax.nn.sigmoid(scores)
    order = jnp.argsort(-probs)
    braw = jnp.pad(boxes[order], ((0, _NP - n), (0, 0)))
    p = jnp.pad(probs[order], (0, _NP - n))[None, :]
    out = pl.pallas_call(
        _nms_kernel,
        out_shape=jax.ShapeDtypeStruct((8, _NP), jnp.float32),
        scratch_shapes=[pltpu.VMEM((1, _NP), jnp.float32)],
    )(braw.T, braw, p)
    return out[:5, :n].T


# speedup vs baseline: 322.9987x; 1.1279x over previous
import jax
import jax.numpy as jnp
from jax.experimental import pallas as pl
from jax.experimental.pallas import tpu as pltpu

_N = 5000
_B = 256
_NP = 5120
_NB = _NP // _B
_THR = 0.65
_STRIDE = 32.0


def _decode_rows(raw):
    cx = raw[0:1, :] * 64.0 + 512.0
    cy = raw[1:2, :] * 64.0 + 512.0
    w = jnp.exp(jnp.clip(raw[2:3, :], -4.0, 4.0)) * _STRIDE
    h = jnp.exp(jnp.clip(raw[3:4, :], -4.0, 4.0)) * _STRIDE
    x1 = cx - w * 0.5
    y1 = cy - h * 0.5
    x2 = cx + w * 0.5
    y2 = cy + h * 0.5
    return x1, y1, x2, y2, (x2 - x1) * (y2 - y1)


def _decode_cols(raw):
    cx = raw[:, 0:1] * 64.0 + 512.0
    cy = raw[:, 1:2] * 64.0 + 512.0
    w = jnp.exp(jnp.clip(raw[:, 2:3], -4.0, 4.0)) * _STRIDE
    h = jnp.exp(jnp.clip(raw[:, 3:4], -4.0, 4.0)) * _STRIDE
    x1 = cx - w * 0.5
    y1 = cy - h * 0.5
    x2 = cx + w * 0.5
    y2 = cy + h * 0.5
    return x1, y1, x2, y2, (x2 - x1) * (y2 - y1)


def _iou(bx1, by1, bx2, by2, ba, x1, y1, x2, y2, area):
    ix1 = jnp.maximum(bx1, x1)
    iy1 = jnp.maximum(by1, y1)
    ix2 = jnp.minimum(bx2, x2)
    iy2 = jnp.minimum(by2, y2)
    inter = jnp.maximum(ix2 - ix1, 0.0) * jnp.maximum(iy2 - iy1, 0.0)
    return inter / (ba + area - inter + 1e-7)


def _nms_kernel(raw_t_ref, raw_ref, p_ref, out_ref, alive_ref):
    raw_t = raw_t_ref[...]
    p = p_ref[...]

    x1, y1, x2, y2, area = _decode_rows(raw_t)
    rowi = jax.lax.broadcasted_iota(jnp.int32, (_B, _B), 0)
    coli = jax.lax.broadcasted_iota(jnp.int32, (_B, _B), 1)
    tri = (rowi < coli).astype(jnp.float32)

    alive_ref[...] = jnp.ones((1, _NP), jnp.float32)

    def block_body(i, carry):
        off = i * _B
        blk = raw_ref[pl.ds(off, _B), :]
        bx1, by1, bx2, by2, ba = _decode_cols(blk)
        blk_t = raw_t_ref[:, pl.ds(off, _B)]
        rx1, ry1, rx2, ry2, rarea = _decode_rows(blk_t)

        mf = (_iou(bx1, by1, bx2, by2, ba,
                   rx1, ry1, rx2, ry2, rarea) > _THR).astype(jnp.float32)
        mf = mf * tri

        blk_alive = alive_ref[:, pl.ds(off, _B)]

        def fix_cond(c):
            return c[1]

        def fix_body(c):
            k, _ = c
            hit = jnp.dot(k, mf, preferred_element_type=jnp.float32)
            k_new = blk_alive * (hit <= 0.0).astype(jnp.float32)
            return k_new, jnp.any(k_new != k)

        k, _ = jax.lax.while_loop(fix_cond, fix_body,
                                  (blk_alive, jnp.bool_(True)))
        alive_ref[:, pl.ds(off, _B)] = k

        def col_body(j, c2):
            joff = j * _B
            ct = raw_t_ref[:, pl.ds(joff, _B)]
            cx1, cy1, cx2, cy2, car = _decode_rows(ct)
            supf = (_iou(bx1, by1, bx2, by2, ba,
                         cx1, cy1, cx2, cy2, car) > _THR).astype(jnp.float32)
            cross = jnp.dot(k, supf, preferred_element_type=jnp.float32)
            alive_ref[:, pl.ds(joff, _B)] = (
                alive_ref[:, pl.ds(joff, _B)]
                * (cross <= 0.0).astype(jnp.float32))
            return c2

        return jax.lax.fori_loop(i + 1, _NB, col_body, carry)

    jax.lax.fori_loop(0, _NB, block_body, 0)

    keep = alive_ref[...]
    zeros = jnp.zeros((3, _NP), jnp.float32)
    out_ref[...] = jnp.concatenate(
        [x1 * keep, y1 * keep, x2 * keep, y2 * keep, p * keep, zeros], axis=0)


def kernel(boxes, scores):
    n = boxes.shape[0]
    probs = jax.nn.sigmoid(scores)
    order = jnp.argsort(-probs)
    braw = jnp.pad(boxes[order], ((0, _NP - n), (0, 0)))
    p = jnp.pad(probs[order], (0, _NP - n))[None, :]
    out = pl.pallas_call(
        _nms_kernel,
        out_shape=jax.ShapeDtypeStruct((8, _NP), jnp.float32),
        scratch_shapes=[pltpu.VMEM((1, _NP), jnp.float32)],
    )(braw.T, braw, p)
    return out[:5, :n].T


# speedup vs baseline: 465.1698x; 1.4402x over previous
import jax
import jax.numpy as jnp
from jax.experimental import pallas as pl
from jax.experimental.pallas import tpu as pltpu

_N = 5000
_B = 512
_NP = 5120
_NB = _NP // _B
_THR = 0.65
_STRIDE = 32.0


def _decode_rows(raw):
    cx = raw[0:1, :] * 64.0 + 512.0
    cy = raw[1:2, :] * 64.0 + 512.0
    w = jnp.exp(jnp.clip(raw[2:3, :], -4.0, 4.0)) * _STRIDE
    h = jnp.exp(jnp.clip(raw[3:4, :], -4.0, 4.0)) * _STRIDE
    x1 = cx - w * 0.5
    y1 = cy - h * 0.5
    x2 = cx + w * 0.5
    y2 = cy + h * 0.5
    return x1, y1, x2, y2, (x2 - x1) * (y2 - y1)


def _decode_cols(raw):
    cx = raw[:, 0:1] * 64.0 + 512.0
    cy = raw[:, 1:2] * 64.0 + 512.0
    w = jnp.exp(jnp.clip(raw[:, 2:3], -4.0, 4.0)) * _STRIDE
    h = jnp.exp(jnp.clip(raw[:, 3:4], -4.0, 4.0)) * _STRIDE
    x1 = cx - w * 0.5
    y1 = cy - h * 0.5
    x2 = cx + w * 0.5
    y2 = cy + h * 0.5
    return x1, y1, x2, y2, (x2 - x1) * (y2 - y1)


def _iou(bx1, by1, bx2, by2, ba, x1, y1, x2, y2, area):
    ix1 = jnp.maximum(bx1, x1)
    iy1 = jnp.maximum(by1, y1)
    ix2 = jnp.minimum(bx2, x2)
    iy2 = jnp.minimum(by2, y2)
    inter = jnp.maximum(ix2 - ix1, 0.0) * jnp.maximum(iy2 - iy1, 0.0)
    return inter / (ba + area - inter + 1e-7)


def _nms_kernel(raw_t_ref, raw_ref, out_ref, alive_ref):
    raw_t = raw_t_ref[...]
    p = raw_t[4:5, :]

    x1, y1, x2, y2, _ = _decode_rows(raw_t)
    rowi = jax.lax.broadcasted_iota(jnp.int32, (_B, _B), 0)
    coli = jax.lax.broadcasted_iota(jnp.int32, (_B, _B), 1)
    tri = (rowi < coli).astype(jnp.float32)

    alive_ref[...] = jnp.ones((1, _NP), jnp.float32)

    def block_body(i, carry):
        off = i * _B
        blk = raw_ref[pl.ds(off, _B), :]
        bx1, by1, bx2, by2, ba = _decode_cols(blk)
        blk_t = raw_t_ref[:, pl.ds(off, _B)]
        rx1, ry1, rx2, ry2, rarea = _decode_rows(blk_t)

        mf = (_iou(bx1, by1, bx2, by2, ba,
                   rx1, ry1, rx2, ry2, rarea) > _THR).astype(jnp.float32)
        mf = mf * tri

        blk_alive = alive_ref[:, pl.ds(off, _B)]

        def fix_cond(c):
            return c[1]

        def fix_body(c):
            k, _ = c
            hit = jnp.dot(k, mf, preferred_element_type=jnp.float32)
            k_new = blk_alive * (hit <= 0.0).astype(jnp.float32)
            return k_new, jnp.any(k_new != k)

        k, _ = jax.lax.while_loop(fix_cond, fix_body,
                                  (blk_alive, jnp.bool_(True)))
        alive_ref[:, pl.ds(off, _B)] = k

        def col_body(j, c2):
            joff = j * _B
            ct = raw_t_ref[:, pl.ds(joff, _B)]
            cx1, cy1, cx2, cy2, car = _decode_rows(ct)
            supf = (_iou(bx1, by1, bx2, by2, ba,
                         cx1, cy1, cx2, cy2, car) > _THR).astype(jnp.float32)
            cross = jnp.dot(k, supf, preferred_element_type=jnp.float32)
            alive_ref[:, pl.ds(joff, _B)] = (
                alive_ref[:, pl.ds(joff, _B)]
                * (cross <= 0.0).astype(jnp.float32))
            return c2

        return jax.lax.fori_loop(i + 1, _NB, col_body, carry)

    jax.lax.fori_loop(0, _NB, block_body, 0)

    keep = alive_ref[...]
    zeros = jnp.zeros((3, _NP), jnp.float32)
    out_ref[...] = jnp.concatenate(
        [x1 * keep, y1 * keep, x2 * keep, y2 * keep, p * keep, zeros], axis=0)


def kernel(boxes, scores):
    n = boxes.shape[0]
    probs = jax.nn.sigmoid(scores)
    order = jnp.argsort(-probs)
    cat = jnp.concatenate([boxes, probs[:, None]], axis=1)
    s5 = jnp.pad(cat[order], ((0, _NP - n), (0, 0)))
    out = pl.pallas_call(
        _nms_kernel,
        out_shape=jax.ShapeDtypeStruct((8, _NP), jnp.float32),
        scratch_shapes=[pltpu.VMEM((1, _NP), jnp.float32)],
    )(s5.T, s5)
    return out[:5, :n].T


# speedup vs baseline: 467.7166x; 1.0055x over previous
import jax
import jax.numpy as jnp
from jax.experimental import pallas as pl
from jax.experimental.pallas import tpu as pltpu

_N = 5000
_B = 512
_NP = 5120
_NB = _NP // _B
_THR = 0.65
_STRIDE = 32.0


def _decode_rows(raw):
    cx = raw[0:1, :] * 64.0 + 512.0
    cy = raw[1:2, :] * 64.0 + 512.0
    w = jnp.exp(jnp.clip(raw[2:3, :], -4.0, 4.0)) * _STRIDE
    h = jnp.exp(jnp.clip(raw[3:4, :], -4.0, 4.0)) * _STRIDE
    x1 = cx - w * 0.5
    y1 = cy - h * 0.5
    x2 = cx + w * 0.5
    y2 = cy + h * 0.5
    return x1, y1, x2, y2, (x2 - x1) * (y2 - y1)


def _decode_cols(raw):
    cx = raw[:, 0:1] * 64.0 + 512.0
    cy = raw[:, 1:2] * 64.0 + 512.0
    w = jnp.exp(jnp.clip(raw[:, 2:3], -4.0, 4.0)) * _STRIDE
    h = jnp.exp(jnp.clip(raw[:, 3:4], -4.0, 4.0)) * _STRIDE
    x1 = cx - w * 0.5
    y1 = cy - h * 0.5
    x2 = cx + w * 0.5
    y2 = cy + h * 0.5
    return x1, y1, x2, y2, (x2 - x1) * (y2 - y1)


def _iou(bx1, by1, bx2, by2, ba, x1, y1, x2, y2, area):
    ix1 = jnp.maximum(bx1, x1)
    iy1 = jnp.maximum(by1, y1)
    ix2 = jnp.minimum(bx2, x2)
    iy2 = jnp.minimum(by2, y2)
    inter = jnp.maximum(ix2 - ix1, 0.0) * jnp.maximum(iy2 - iy1, 0.0)
    return inter / (ba + area - inter + 1e-7)


def _nms_kernel(raw_t_ref, raw_ref, out_ref, alive_ref, coords_ref):
    raw_t = raw_t_ref[...]
    p = raw_t[4:5, :]

    x1, y1, x2, y2, area = _decode_rows(raw_t)
    coords_ref[...] = jnp.concatenate(
        [x1, y1, x2, y2, area, p, p, p], axis=0)
    rowi = jax.lax.broadcasted_iota(jnp.int32, (_B, _B), 0)
    coli = jax.lax.broadcasted_iota(jnp.int32, (_B, _B), 1)
    tri = (rowi < coli).astype(jnp.float32)

    alive_ref[...] = jnp.ones((1, _NP), jnp.float32)

    def block_body(i, carry):
        off = i * _B
        blk = raw_ref[pl.ds(off, _B), :]
        bx1, by1, bx2, by2, ba = _decode_cols(blk)
        blk_c = coords_ref[:, pl.ds(off, _B)]
        rx1, ry1 = blk_c[0:1, :], blk_c[1:2, :]
        rx2, ry2 = blk_c[2:3, :], blk_c[3:4, :]
        rarea = blk_c[4:5, :]

        mf = (_iou(bx1, by1, bx2, by2, ba,
                   rx1, ry1, rx2, ry2, rarea) > _THR).astype(jnp.float32)
        mf = mf * tri

        blk_alive = alive_ref[:, pl.ds(off, _B)]

        def fix_cond(c):
            return c[1]

        def fix_body(c):
            k, _ = c
            hit = jnp.dot(k, mf, preferred_element_type=jnp.float32)
            k_new = blk_alive * (hit <= 0.0).astype(jnp.float32)
            return k_new, jnp.any(k_new != k)

        k, _ = jax.lax.while_loop(fix_cond, fix_body,
                                  (blk_alive, jnp.bool_(True)))
        alive_ref[:, pl.ds(off, _B)] = k

        def col_body(j, c2):
            joff = j * _B
            ct = coords_ref[:, pl.ds(joff, _B)]
            cx1, cy1 = ct[0:1, :], ct[1:2, :]
            cx2, cy2 = ct[2:3, :], ct[3:4, :]
            car = ct[4:5, :]
            supf = (_iou(bx1, by1, bx2, by2, ba,
                         cx1, cy1, cx2, cy2, car) > _THR).astype(jnp.float32)
            cross = jnp.dot(k, supf, preferred_element_type=jnp.float32)
            alive_ref[:, pl.ds(joff, _B)] = (
                alive_ref[:, pl.ds(joff, _B)]
                * (cross <= 0.0).astype(jnp.float32))
            return c2

        return jax.lax.fori_loop(i + 1, _NB, col_body, carry)

    jax.lax.fori_loop(0, _NB, block_body, 0)

    keep = alive_ref[...]
    zeros = jnp.zeros((3, _NP), jnp.float32)
    out_ref[...] = jnp.concatenate(
        [x1 * keep, y1 * keep, x2 * keep, y2 * keep, p * keep, zeros], axis=0)


def kernel(boxes, scores):
    n = boxes.shape[0]
    probs = jax.nn.sigmoid(scores)
    order = jnp.argsort(-probs)
    cat = jnp.concatenate([boxes, probs[:, None]], axis=1)
    s5 = jnp.pad(cat[order], ((0, _NP - n), (0, 0)))
    out = pl.pallas_call(
        _nms_kernel,
        out_shape=jax.ShapeDtypeStruct((8, _NP), jnp.float32),
        scratch_shapes=[pltpu.VMEM((1, _NP), jnp.float32),
                        pltpu.VMEM((8, _NP), jnp.float32)],
    )(s5.T, s5)
    return out[:5, :n].T
